# baseline (speedup 1.0000x reference)
import jax
import jax.numpy as jnp
from jax.experimental import pallas as pl
from jax.experimental.pallas import tpu as pltpu

_EPS = 1e-08
_B, _S, _D = 4, 2048, 768
_NUM_CLASSES_SEEN = 10
_TOP_K = 3
_N_DESC = 5
_NSTEPS = 2
_ROWS = (_B * _S) // _NSTEPS


def _fused_body(x_ref, ck_ref, desc_ref, w_ref, g_ref, b_ref, t_ref,
                out_ref, acc_ref, dp_ref):
    i = pl.program_id(0)

    if _ROWS <= _S:
        blks_per_batch = _S // _ROWS
        partial = jnp.sum(x_ref[...], axis=0, keepdims=True)
        b = i // blks_per_batch

        @pl.when(i % blks_per_batch == 0)
        def _init():
            acc_ref[pl.ds(b, 1), :] = partial

        @pl.when(i % blks_per_batch != 0)
        def _accum():
            acc_ref[pl.ds(b, 1), :] = acc_ref[pl.ds(b, 1), :] + partial
    else:
        bpb = _ROWS // _S
        partial = jnp.sum(x_ref[...].reshape(bpb, _S, _D), axis=1)
        for k in range(_NSTEPS):
            @pl.when(i == k)
            def _store(k=k):
                acc_ref[k * bpb:(k + 1) * bpb, :] = partial

    @pl.when(i == 0)
    def _dprime():
        dp_ref[0:_N_DESC, :] = jax.lax.dot_general(
            desc_ref[...], w_ref[...], (((1,), (1,)), ((), ())),
            preferred_element_type=jnp.float32)

    @pl.when(i == _NSTEPS - 1)
    def _epilogue():
        mean = acc_ref[0:_B, :] * (1.0 / _S)
        xnorm = jnp.sqrt(jnp.sum(mean * mean, axis=1, keepdims=True))
        xn = mean / jnp.maximum(xnorm, _EPS)

        ck = ck_ref[...]
        cknorm = jnp.sqrt(jnp.sum(ck * ck, axis=1, keepdims=True))
        ckn = ck / jnp.maximum(cknorm, _EPS)

        sims = jax.lax.dot_general(
            xn, ckn, (((1,), (1,)), ((), ())),
            preferred_element_type=jnp.float32)

        t = t_ref[0, 0]

        col = jax.lax.broadcasted_iota(jnp.int32, (_B, _NUM_CLASSES_SEEN), 1)
        s = sims
        vals = []
        idxs = []
        for _ in range(_TOP_K):
            m = jnp.max(s, axis=1, keepdims=True)
            idx = jnp.min(jnp.where(s >= m, col, _NUM_CLASSES_SEEN + 1),
                          axis=1, keepdims=True)
            vals.append(m)
            idxs.append(idx)
            s = jnp.where(col == idx, -jnp.inf, s)

        exps = [jnp.exp((v - vals[0]) * t) for v in vals]
        denom = exps[0] + exps[1] + exps[2]
        ws = [e / denom for e in exps]

        dcol = jax.lax.broadcasted_iota(jnp.int32, (_B, _N_DESC), 1)
        dw = jnp.zeros((_B, _N_DESC), jnp.float32)
        for k in range(_TOP_K):
            didx = jax.lax.rem(idxs[k], _N_DESC)
            dw = dw + jnp.where(dcol == didx, ws[k], 0.0)

        proj = jax.lax.dot_general(
            dw, dp_ref[0:_N_DESC, :], (((1,), (0,)), ((), ())),
            preferred_element_type=jnp.float32)

        mu = jnp.mean(proj, axis=1, keepdims=True)
        ctr = proj - mu
        var = jnp.mean(ctr * ctr, axis=1, keepdims=True)
        ln = ctr * jax.lax.rsqrt(var + 1e-05) * g_ref[...] + b_ref[...]

        out_ref[:, 0, :] = ln


@jax.jit
def kernel(x_embed, prompt_key, task_key, desc_emb, W_proj, ln_gamma,
           ln_beta, temperature):
    del task_key
    xf = x_embed.reshape(_B * _S, _D)
    ck = prompt_key[:_NUM_CLASSES_SEEN]
    gamma = ln_gamma.reshape(1, _D)
    beta = ln_beta.reshape(1, _D)
    temp = temperature.reshape(1, 1)

    out = pl.pallas_call(
        _fused_body,
        grid=(_NSTEPS,),
        in_specs=[
            pl.BlockSpec((_ROWS, _D), lambda i: (i, 0)),
            pl.BlockSpec((_NUM_CLASSES_SEEN, _D), lambda i: (0, 0)),
            pl.BlockSpec((_N_DESC, _D), lambda i: (0, 0)),
            pl.BlockSpec((_D, _D), lambda i: (0, 0)),
            pl.BlockSpec((1, _D), lambda i: (0, 0)),
            pl.BlockSpec((1, _D), lambda i: (0, 0)),
            pl.BlockSpec((1, 1), lambda i: (0, 0)),
        ],
        out_specs=pl.BlockSpec((_B, 1, _D), lambda i: (0, 0, 0)),
        out_shape=jax.ShapeDtypeStruct((_B, 1, _D), jnp.float32),
        scratch_shapes=[pltpu.VMEM((8, _D), jnp.float32),
                        pltpu.VMEM((8, _D), jnp.float32)],
    )(xf, ck, desc_emb, W_proj, gamma, beta, temp)
    return out


# speedup vs baseline: 1.0215x; 1.0215x over previous
import jax
import jax.numpy as jnp
from jax.experimental import pallas as pl
from jax.experimental.pallas import tpu as pltpu

_EPS = 1e-08
_B, _S, _D = 4, 2048, 768
_NUM_CLASSES_SEEN = 10
_TOP_K = 3
_N_DESC = 5
_NSTEPS = 4
_ROWS = (_B * _S) // (2 * _NSTEPS)


def _fused_body(xa_ref, xb_ref, ck_ref, desc_ref, w_ref, g_ref, b_ref,
                t_ref, out_ref, acc_ref, dp_ref):
    i = pl.program_id(0)

    half_rows = (_B * _S) // 2
    blks_per_batch = _S // _ROWS
    pa = jnp.sum(xa_ref[...], axis=0, keepdims=True)
    pb = jnp.sum(xb_ref[...], axis=0, keepdims=True)
    ba = i // blks_per_batch
    bb = (_B // 2) + i // blks_per_batch

    @pl.when(i % blks_per_batch == 0)
    def _init():
        acc_ref[pl.ds(ba, 1), :] = pa
        acc_ref[pl.ds(bb, 1), :] = pb

    @pl.when(i % blks_per_batch != 0)
    def _accum():
        acc_ref[pl.ds(ba, 1), :] = acc_ref[pl.ds(ba, 1), :] + pa
        acc_ref[pl.ds(bb, 1), :] = acc_ref[pl.ds(bb, 1), :] + pb

    @pl.when(i == 0)
    def _dprime():
        dp_ref[0:_N_DESC, :] = jax.lax.dot_general(
            desc_ref[...], w_ref[...], (((1,), (1,)), ((), ())),
            preferred_element_type=jnp.float32)

    @pl.when(i == _NSTEPS - 1)
    def _epilogue():
        mean = acc_ref[0:_B, :] * (1.0 / _S)
        xnorm = jnp.sqrt(jnp.sum(mean * mean, axis=1, keepdims=True))
        xn = mean / jnp.maximum(xnorm, _EPS)

        ck = ck_ref[...]
        cknorm = jnp.sqrt(jnp.sum(ck * ck, axis=1, keepdims=True))
        ckn = ck / jnp.maximum(cknorm, _EPS)

        sims = jax.lax.dot_general(
            xn, ckn, (((1,), (1,)), ((), ())),
            preferred_element_type=jnp.float32)

        t = t_ref[0, 0]

        col = jax.lax.broadcasted_iota(jnp.int32, (_B, _NUM_CLASSES_SEEN), 1)
        s = sims
        vals = []
        idxs = []
        for _ in range(_TOP_K):
            m = jnp.max(s, axis=1, keepdims=True)
            idx = jnp.min(jnp.where(s >= m, col, _NUM_CLASSES_SEEN + 1),
                          axis=1, keepdims=True)
            vals.append(m)
            idxs.append(idx)
            s = jnp.where(col == idx, -jnp.inf, s)

        exps = [jnp.exp((v - vals[0]) * t) for v in vals]
        denom = exps[0] + exps[1] + exps[2]
        ws = [e / denom for e in exps]

        dcol = jax.lax.broadcasted_iota(jnp.int32, (_B, _N_DESC), 1)
        dw = jnp.zeros((_B, _N_DESC), jnp.float32)
        for k in range(_TOP_K):
            didx = jax.lax.rem(idxs[k], _N_DESC)
            dw = dw + jnp.where(dcol == didx, ws[k], 0.0)

        proj = jax.lax.dot_general(
            dw, dp_ref[0:_N_DESC, :], (((1,), (0,)), ((), ())),
            preferred_element_type=jnp.float32)

        mu = jnp.mean(proj, axis=1, keepdims=True)
        ctr = proj - mu
        var = jnp.mean(ctr * ctr, axis=1, keepdims=True)
        ln = ctr * jax.lax.rsqrt(var + 1e-05) * g_ref[...] + b_ref[...]

        out_ref[:, 0, :] = ln


@jax.jit
def kernel(x_embed, prompt_key, task_key, desc_emb, W_proj, ln_gamma,
           ln_beta, temperature):
    del task_key
    xf = x_embed.reshape(_B * _S, _D)
    ck = prompt_key[:_NUM_CLASSES_SEEN]
    gamma = ln_gamma.reshape(1, _D)
    beta = ln_beta.reshape(1, _D)
    temp = temperature.reshape(1, 1)

    out = pl.pallas_call(
        _fused_body,
        grid=(_NSTEPS,),
        in_specs=[
            pl.BlockSpec((_ROWS, _D), lambda i: (i, 0)),
            pl.BlockSpec((_ROWS, _D), lambda i: (i + _NSTEPS, 0)),
            pl.BlockSpec((_NUM_CLASSES_SEEN, _D), lambda i: (0, 0)),
            pl.BlockSpec((_N_DESC, _D), lambda i: (0, 0)),
            pl.BlockSpec((_D, _D), lambda i: (0, 0)),
            pl.BlockSpec((1, _D), lambda i: (0, 0)),
            pl.BlockSpec((1, _D), lambda i: (0, 0)),
            pl.BlockSpec((1, 1), lambda i: (0, 0)),
        ],
        out_specs=pl.BlockSpec((_B, 1, _D), lambda i: (0, 0, 0)),
        out_shape=jax.ShapeDtypeStruct((_B, 1, _D), jnp.float32),
        scratch_shapes=[pltpu.VMEM((8, _D), jnp.float32),
                        pltpu.VMEM((8, _D), jnp.float32)],
    )(xf, xf, ck, desc_emb, W_proj, gamma, beta, temp)
    return out
